# Initial kernel scaffold; baseline (speedup 1.0000x reference)
import jax
import jax.numpy as jnp
from jax.experimental import pallas as pl


def kernel(positions, emb_table, fw1_1, fw2_1, tpw1, scw1, fw1_2, fw2_2, tpw2, scw2, fw1_3, fw2_3, tpw3, scw3, atom_types):
    raise NotImplementedError("write your pallas kernel here")


# speedup vs baseline: 1.1011x; 1.1011x over previous
import functools

import numpy as np
import jax
import jax.numpy as jnp
from jax.experimental import pallas as pl
from jax.experimental.pallas import tpu as pltpu

_CUTOFF = 5.0
_NB = 128
_BLK_N = 16
_BLK_E = _BLK_N * _NB
_SQ23 = float(np.sqrt(2.0 / _CUTOFF))
_S3 = float(np.sqrt(3.0))
_S15 = float(np.sqrt(15.0))
_S5H = float(np.sqrt(5.0) / 2.0)
_PI = float(np.pi)


def _gate_expand_np():
    m = np.zeros((16, 64), np.float32)
    off = 0
    for j in range(4):
        m[j, off + j * 3: off + (j + 1) * 3] = 1.0
    off = 12
    for j in range(4):
        m[4 + j, off + j * 3: off + (j + 1) * 3] = 1.0
    off = 24
    for j in range(4):
        m[8 + j, off + j * 5: off + (j + 1) * 5] = 1.0
    off = 44
    for j in range(4):
        m[12 + j, off + j * 5: off + (j + 1) * 5] = 1.0
    return m.T.copy()


def _feat_body(evm_ref, out_ref):
    ev = evm_ref[0:3, :]
    maskf = evm_ref[3:4, :]
    tes = evm_ref[4:8, :]
    ted = evm_ref[8:12, :]
    r2 = jnp.sum(ev * ev, axis=0, keepdims=True)
    r = jnp.sqrt(r2)
    rs = jnp.maximum(r, 1e-9)
    inv = 1.0 / rs
    x = r * (1.0 / _CUTOFF)
    s1 = jnp.sin(_PI * x)
    c1 = jnp.cos(_PI * x)
    x3 = x * x * x
    x6 = x3 * x3
    u = 1.0 + x6 * (-28.0 + x * (48.0 - 21.0 * x))
    k = (_SQ23 * u * maskf) * inv
    bes = [s1 * k]
    sn, cn = s1, c1
    for _ in range(9):
        sn, cn = sn * c1 + cn * s1, cn * c1 - sn * s1
        bes.append(sn * k)
    v = ev * inv
    vx = v[0:1, :]
    vy = v[1:2, :]
    vz = v[2:3, :]
    sh = [
        jnp.ones_like(vx),
        _S3 * vy, _S3 * vz, _S3 * vx,
        _S15 * vx * vy, _S15 * vy * vz, _S5H * (3.0 * vz * vz - 1.0),
        _S15 * vx * vz, (_S15 * 0.5) * (vx * vx - vy * vy),
    ]
    pad = jnp.zeros((5, _BLK_E), jnp.float32)
    out_ref[...] = jnp.concatenate(
        [ted * maskf, tes * maskf] + bes + sh + [pad], axis=0)


def _seg_mat():
    e = jax.lax.broadcasted_iota(jnp.int32, (_BLK_E, _BLK_N), 0)
    n = jax.lax.broadcasted_iota(jnp.int32, (_BLK_E, _BLK_N), 1)
    return jnp.where((e >> 7) == n, 1.0, 0.0).astype(jnp.float32)


def _edge_mlp(feat_ref, fw1t_ref, fw2t_ref):
    ee = feat_ref[0:18, :]
    h = jax.nn.relu(jnp.dot(fw1t_ref[...], ee,
                            preferred_element_type=jnp.float32))
    return jnp.dot(fw2t_ref[...], h, preferred_element_type=jnp.float32)


def _conv_gate_body(feat_ref, xg_ref, x_ref, fw1t_ref, fw2t_ref,
                    tpwft_ref, scwt_ref, expt_ref, out_ref, *, dp, first):
    w = _edge_mlp(feat_ref, fw1t_ref, fw2t_ref)
    xgt = xg_ref[...].T
    f = xgt * w
    st = _seg_mat()
    qs = []
    for s in range(9):
        t = f * feat_ref[18 + s: 19 + s, :]
        qs.append(jnp.dot(t, st, preferred_element_type=jnp.float32))
    p = jnp.concatenate(qs, axis=0)
    conv = jnp.dot(tpwft_ref[...], p, preferred_element_type=jnp.float32) * 0.25
    xt = x_ref[...].T
    conv = conv + jnp.dot(scwt_ref[...], xt[0:dp, :],
                          preferred_element_type=jnp.float32)
    sc0 = jax.nn.relu(conv[0:4, :])
    if first:
        sc0 = sc0 + xt[0:4, :]
    sc1 = jnp.abs(conv[4:12, :])
    g = jnp.concatenate([
        jax.nn.relu(conv[12:16, :]), jnp.tanh(conv[16:20, :]),
        jax.nn.relu(conv[20:24, :]), jnp.tanh(conv[24:28, :]),
    ], axis=0)
    gex = jnp.dot(expt_ref[...], g, preferred_element_type=jnp.float32)
    gated = conv[28:92, :] * gex
    out = jnp.concatenate(
        [sc0, sc1, gated, jnp.zeros((4, _BLK_N), jnp.float32)], axis=0)
    if not first:
        out = out + xt
    out_ref[...] = out.T


def _conv_final_body(feat_ref, xg_ref, x_ref, fw1t_ref, fw2t_ref,
                     t3t_ref, scw3t_ref, out_ref):
    w = _edge_mlp(feat_ref, fw1t_ref, fw2t_ref)
    xgt = xg_ref[...].T
    f = xgt * w
    z = jnp.dot(t3t_ref[...], f, preferred_element_type=jnp.float32)
    part_e = jnp.sum(z * feat_ref[18:27, :])
    xt = x_ref[...].T
    part_x = jnp.sum(jnp.dot(scw3t_ref[...], xt,
                             preferred_element_type=jnp.float32))
    @pl.when(pl.program_id(0) == 0)
    def _():
        out_ref[...] = jnp.zeros_like(out_ref)
    out_ref[...] += jnp.reshape(part_e * 0.25 + part_x, (1, 1))


def _feat_kernel(evmt):
    e = evmt.shape[1]
    grid = e // _BLK_E
    return pl.pallas_call(
        _feat_body,
        grid=(grid,),
        in_specs=[pl.BlockSpec((16, _BLK_E), lambda i: (0, i))],
        out_specs=pl.BlockSpec((32, _BLK_E), lambda i: (0, i)),
        out_shape=jax.ShapeDtypeStruct((32, e), jnp.float32),
    )(evmt)


def _conv_layer(feats, xg, x, fw1t, fw2t, tpwft, scwt, expt, *, dp, first):
    n = x.shape[0]
    grid = n // _BLK_N
    return pl.pallas_call(
        functools.partial(_conv_gate_body, dp=dp, first=first),
        grid=(grid,),
        in_specs=[
            pl.BlockSpec((32, _BLK_E), lambda i: (0, i)),
            pl.BlockSpec((_BLK_E, dp), lambda i: (i, 0)),
            pl.BlockSpec((_BLK_N, 80), lambda i: (i, 0)),
            pl.BlockSpec(fw1t.shape, lambda i: (0, 0)),
            pl.BlockSpec(fw2t.shape, lambda i: (0, 0)),
            pl.BlockSpec(tpwft.shape, lambda i: (0, 0)),
            pl.BlockSpec(scwt.shape, lambda i: (0, 0)),
            pl.BlockSpec(expt.shape, lambda i: (0, 0)),
        ],
        out_specs=pl.BlockSpec((_BLK_N, 80), lambda i: (i, 0)),
        out_shape=jax.ShapeDtypeStruct((n, 80), jnp.float32),
    )(feats, xg, x, fw1t, fw2t, tpwft, scwt, expt)


def _conv_final(feats, xg, x, fw1t, fw2t, t3t, scw3t):
    n = x.shape[0]
    grid = n // _BLK_N
    out = pl.pallas_call(
        _conv_final_body,
        grid=(grid,),
        in_specs=[
            pl.BlockSpec((32, _BLK_E), lambda i: (0, i)),
            pl.BlockSpec((_BLK_E, 80), lambda i: (i, 0)),
            pl.BlockSpec((_BLK_N, 80), lambda i: (i, 0)),
            pl.BlockSpec(fw1t.shape, lambda i: (0, 0)),
            pl.BlockSpec(fw2t.shape, lambda i: (0, 0)),
            pl.BlockSpec(t3t.shape, lambda i: (0, 0)),
            pl.BlockSpec(scw3t.shape, lambda i: (0, 0)),
        ],
        out_specs=pl.BlockSpec((1, 1), lambda i: (0, 0)),
        out_shape=jax.ShapeDtypeStruct((1, 1), jnp.float32),
        compiler_params=pltpu.CompilerParams(
            dimension_semantics=("arbitrary",)),
    )(feats, xg, x, fw1t, fw2t, t3t, scw3t)
    return out[0, 0]


def _radius_graph_ref(pos):
    n = pos.shape[0]
    sq = (pos * pos).sum(-1)
    d2 = sq[:, None] + sq[None, :] - 2.0 * (pos @ pos.T)
    neg, cols = jax.lax.top_k(-d2, _NB)
    rows = jnp.broadcast_to(jnp.arange(n, dtype=jnp.int32)[:, None], (n, _NB))
    cols = cols.astype(jnp.int32)
    keep = ((-neg) < _CUTOFF * _CUTOFF) & (rows != cols)
    return cols.reshape(-1), rows.reshape(-1), keep.reshape(-1)


def kernel(positions, emb_table, fw1_1, fw2_1, tpw1, scw1, fw1_2, fw2_2, tpw2,
           scw2, fw1_3, fw2_3, tpw3, scw3, atom_types):
    src, dst, edge_mask = _radius_graph_ref(positions)
    type_embed = emb_table[atom_types]
    e = src.shape[0]
    edge_vec = positions[dst] - positions[src]
    evmt = jnp.concatenate([
        edge_vec.T,
        edge_mask[None, :].astype(jnp.float32),
        type_embed[src].T,
        type_embed[dst].T,
        jnp.zeros((4, e), jnp.float32),
    ], axis=0)

    feats = _feat_kernel(evmt)

    expt = jnp.asarray(_gate_expand_np())
    x0 = jnp.pad(type_embed, ((0, 0), (0, 76)))

    def tpw_flat_t(tpw):
        t = jnp.pad(tpw, ((0, 80 - tpw.shape[0]), (0, 0), (0, 0)))
        return t.transpose(2, 1, 0).reshape(tpw.shape[2], 9 * 80)

    def tpw_flat_t16(tpw):
        t = jnp.pad(tpw, ((0, 16 - tpw.shape[0]), (0, 0), (0, 0)))
        return t.transpose(2, 1, 0).reshape(tpw.shape[2], 9 * 16)

    tpwf1t = tpw_flat_t16(tpw1)
    tpwf2t = tpw_flat_t(tpw2)
    fw2_1t = jnp.pad(fw2_1, ((0, 0), (0, 12))).T
    fw2_2t = jnp.pad(fw2_2, ((0, 0), (0, 4))).T
    fw2_3t = jnp.pad(fw2_3, ((0, 0), (0, 4))).T
    scw1t = scw1.T[:, 0:16]
    scw1t = jnp.pad(scw1.T, ((0, 0), (0, 12)))
    scw2t = jnp.pad(scw2.T, ((0, 0), (0, 4)))
    scw3t = jnp.pad(scw3.T, ((0, 0), (0, 4)))
    t3t = jnp.pad(tpw3[:, :, 0], ((0, 4), (0, 0))).T

    xg1 = x0[:, 0:16][src]
    x1 = _conv_layer(feats, xg1, x0, fw1_1.T, fw2_1t, tpwf1t, scw1t,
                     expt, dp=16, first=True)
    xg2 = x1[src]
    x2 = _conv_layer(feats, xg2, x1, fw1_2.T, fw2_2t, tpwf2t, scw2t,
                     expt, dp=80, first=False)
    xg3 = x2[src]
    return _conv_final(feats, xg3, x2, fw1_3.T, fw2_3t, t3t, scw3t)
